# baseline (speedup 1.0000x reference)
import functools

import jax
import jax.numpy as jnp
from jax import lax
from jax.experimental import pallas as pl
from jax.experimental.pallas import tpu as pltpu
from jax.experimental.pallas import tpu_sc as plsc

B = 16384
D = 32
NC = 2
NS = 16
NW = NC * NS
ROWS_W = B // NW
CHUNK = 128
NCHUNK = ROWS_W // CHUNK


def _sc_gather4(uidx, iidx, t_umlp, t_imlp, t_umf, t_imf):
    mesh = plsc.VectorSubcoreMesh(
        core_axis_name="c", subcore_axis_name="s", num_cores=NC, num_subcores=NS
    )
    out_t = [jax.ShapeDtypeStruct((NW, NCHUNK, CHUNK, D), jnp.float32)
             for _ in range(4)]

    @functools.partial(
        pl.kernel,
        out_type=out_t,
        mesh=mesh,
        compiler_params=pltpu.CompilerParams(use_tc_tiling_on_sc=False),
        scratch_types=[
            pltpu.VMEM((NCHUNK, CHUNK), jnp.int32),
            pltpu.VMEM((NCHUNK, CHUNK), jnp.int32),
            pltpu.VMEM((NCHUNK, CHUNK, D), jnp.float32),
            pltpu.VMEM((NCHUNK, CHUNK, D), jnp.float32),
            pltpu.VMEM((NCHUNK, CHUNK, D), jnp.float32),
            pltpu.VMEM((NCHUNK, CHUNK, D), jnp.float32),
            pltpu.SemaphoreType.DMA,
        ],
    )
    def gather_kernel(uidx_hbm, iidx_hbm, umlp_hbm, imlp_hbm, umf_hbm, imf_hbm,
                      o_umlp, o_imlp, o_umf, o_imf,
                      uidx_v, iidx_v, b_umlp, b_imlp, b_umf, b_imf, sem):
        wid = lax.axis_index("s") * NC + lax.axis_index("c")
        pltpu.sync_copy(uidx_hbm.at[wid], uidx_v)
        pltpu.sync_copy(iidx_hbm.at[wid], iidx_v)
        copies = []
        for j in range(NCHUNK):
            copies.append(pltpu.async_copy(umlp_hbm.at[uidx_v.at[j]], b_umlp.at[j], sem))
            copies.append(pltpu.async_copy(imlp_hbm.at[iidx_v.at[j]], b_imlp.at[j], sem))
            copies.append(pltpu.async_copy(umf_hbm.at[uidx_v.at[j]], b_umf.at[j], sem))
            copies.append(pltpu.async_copy(imf_hbm.at[iidx_v.at[j]], b_imf.at[j], sem))
        for c in copies:
            c.wait()
        pltpu.sync_copy(b_umlp, o_umlp.at[wid])
        pltpu.sync_copy(b_imlp, o_imlp.at[wid])
        pltpu.sync_copy(b_umf, o_umf.at[wid])
        pltpu.sync_copy(b_imf, o_imf.at[wid])

    u3 = uidx.reshape(NW, NCHUNK, CHUNK)
    i3 = iidx.reshape(NW, NCHUNK, CHUNK)
    outs = gather_kernel(u3, i3, t_umlp, t_imlp, t_umf, t_imf)
    return [o.reshape(B, D) for o in outs]


BLK = 2048


def _mlp_body(ue, ie, um, im, w1a, w1b, b1, w2, b2, w3, b3, wah, wam, ba, out):
    h = jnp.dot(ue[...], w1a[...], preferred_element_type=jnp.float32)
    h = h + jnp.dot(ie[...], w1b[...], preferred_element_type=jnp.float32)
    h = jnp.maximum(h + b1[...], 0.0)
    h = jnp.maximum(jnp.dot(h, w2[...], preferred_element_type=jnp.float32) + b2[...], 0.0)
    h = jnp.maximum(jnp.dot(h, w3[...], preferred_element_type=jnp.float32) + b3[...], 0.0)
    mf = um[...] * im[...]
    logit = (jnp.sum(h * wah[...], axis=1, keepdims=True)
             + jnp.sum(mf * wam[...], axis=1, keepdims=True) + ba[...])
    out[...] = 1.0 / (1.0 + jnp.exp(-logit))


def _tc_mlp(ue, ie, um, im, W1, b1, W2, b2, W3, b3, Wa, ba):
    w1a = W1[:, :D].T
    w1b = W1[:, D:].T
    w2 = W2.T
    w3 = W3.T
    wah = Wa[:, :8]
    wam = Wa[:, 8:]
    row = lambda i: (i, 0)
    full = lambda i: (0, 0)
    grid = B // BLK
    return pl.pallas_call(
        _mlp_body,
        grid=(grid,),
        in_specs=[
            pl.BlockSpec((BLK, D), row),
            pl.BlockSpec((BLK, D), row),
            pl.BlockSpec((BLK, D), row),
            pl.BlockSpec((BLK, D), row),
            pl.BlockSpec((D, D), full),
            pl.BlockSpec((D, D), full),
            pl.BlockSpec((1, D), full),
            pl.BlockSpec((D, 16), full),
            pl.BlockSpec((1, 16), full),
            pl.BlockSpec((16, 8), full),
            pl.BlockSpec((1, 8), full),
            pl.BlockSpec((1, 8), full),
            pl.BlockSpec((1, D), full),
            pl.BlockSpec((1, 1), full),
        ],
        out_specs=pl.BlockSpec((BLK, 1), row),
        out_shape=jax.ShapeDtypeStruct((B, 1), jnp.float32),
    )(ue, ie, um, im, w1a, w1b, b1.reshape(1, D), w2, b2.reshape(1, 16),
      w3, b3.reshape(1, 8), wah, wam, ba.reshape(1, 1))


def kernel(user_indices, item_indices, ratings, emb_user_mlp, emb_item_mlp,
           emb_user_mf, emb_item_mf, W1, b1, W2, b2, W3, b3, Wa, ba):
    ue, ie, um, im = _sc_gather4(
        user_indices.astype(jnp.int32), item_indices.astype(jnp.int32),
        emb_user_mlp, emb_item_mlp, emb_user_mf, emb_item_mf)
    rating = _tc_mlp(ue, ie, um, im, W1, b1, W2, b2, W3, b3, Wa, ba)
    return (rating, ratings.astype(jnp.float32))


# speedup vs baseline: 3.4555x; 3.4555x over previous
import functools

import jax
import jax.numpy as jnp
from jax import lax
from jax.experimental import pallas as pl
from jax.experimental.pallas import tpu as pltpu
from jax.experimental.pallas import tpu_sc as plsc

B = 16384
D = 32
NC = 2
NS = 16
NW = NC * NS
ROWS_W = B // NW
TBK = 4
NBAT = ROWS_W // TBK


def _splat(v, n=16):
    return jnp.broadcast_to(jnp.asarray(v, jnp.int32), (n,))


def _sc_gather4(uidx, iidx, t_umlp, t_imlp, t_umf, t_imf):
    mesh = plsc.VectorSubcoreMesh(
        core_axis_name="c", subcore_axis_name="s", num_cores=NC, num_subcores=NS
    )
    out_t = [jax.ShapeDtypeStruct((NW, 128, 128), jnp.float32) for _ in range(4)]

    @functools.partial(
        pl.kernel,
        out_type=out_t,
        mesh=mesh,
        compiler_params=pltpu.CompilerParams(
            use_tc_tiling_on_sc=True, needs_layout_passes=False),
        scratch_types=[
            pltpu.VMEM((2, ROWS_W // 16, 16), jnp.int32),
            pltpu.VMEM((2, 2, TBK, D, 128), jnp.float32),
            pltpu.VMEM((2, 128, 128), jnp.float32),
            pltpu.SemaphoreType.DMA((2,)),
        ],
    )
    def gather_kernel(uidx_hbm, iidx_hbm, umlp, imlp, umf, imf,
                      o_umlp, o_imlp, o_umf, o_imf,
                      idx_v, pan, outv, sems):
        wid = lax.axis_index("s") * NC + lax.axis_index("c")
        pltpu.sync_copy(uidx_hbm.at[wid], idx_v.at[0])
        pltpu.sync_copy(iidx_hbm.at[wid], idx_v.at[1])
        rows0 = lax.iota(jnp.int32, 16)
        rows1 = rows0 + 16

        def run_pass(p, tabs, outs):
            def sidx(g, l):
                vec = idx_v[p, lax.div(g, 4)]
                lane = 4 * lax.rem(g, 4) + l
                return jnp.max(jnp.where(rows0 == _splat(lane), vec, 0))

            def fire(g, slot):
                for l in range(TBK):
                    s = sidx(g, l)
                    c = pl.multiple_of(jnp.bitwise_and(s, -128), 128)
                    for t in range(2):
                        pltpu.async_copy(
                            tabs[t].at[:, pl.ds(c, 128)], pan.at[slot, t, l],
                            sems.at[slot])

            def drain_extract(g, slot):
                for t in range(2):
                    for l in range(TBK):
                        pltpu.make_async_copy(
                            tabs[t].at[:, pl.ds(0, 128)], pan.at[slot, t, l],
                            sems.at[slot]).wait()
                for l in range(TBK):
                    col = _splat(jnp.bitwise_and(sidx(g, l), 127))
                    c0 = l * D
                    for t in range(2):
                        v0 = plsc.load_gather(
                            pan, [_splat(slot), _splat(t), _splat(l), rows0, col])
                        v1 = plsc.load_gather(
                            pan, [_splat(slot), _splat(t), _splat(l), rows1, col])
                        outv[t, g, pl.ds(c0, 16)] = v0
                        outv[t, g, pl.ds(c0 + 16, 16)] = v1

            fire(0, 0)

            def body(g2, carry):
                g0 = 2 * g2
                fire(g0 + 1, 1)
                drain_extract(g0, 0)

                @pl.when(g2 < NBAT // 2 - 1)
                def _():
                    fire(g0 + 2, 0)

                drain_extract(g0 + 1, 1)
                return carry

            lax.fori_loop(0, NBAT // 2, body, 0)
            for t in range(2):
                pltpu.sync_copy(outv.at[t], outs[t].at[wid])

        run_pass(0, (umlp, umf), (o_umlp, o_umf))
        run_pass(1, (imlp, imf), (o_imlp, o_imf))

    u2 = uidx.reshape(NW, ROWS_W // 16, 16)
    i2 = iidx.reshape(NW, ROWS_W // 16, 16)
    outs = gather_kernel(u2, i2, t_umlp, t_imlp, t_umf, t_imf)
    return [o.reshape(B // 4, 128) for o in outs]


def _blockdiag4(a):
    m, n = a.shape
    out = jnp.zeros((4 * m, 4 * n), jnp.float32)
    for j in range(4):
        out = out.at[j * m:(j + 1) * m, j * n:(j + 1) * n].set(a)
    return out


BLK = 1024


def _mlp_body(xu, xi, xum, xim, w1a, w1b, b1, w2, b2, w3, b3, wah, wam, ba, out):
    h = jnp.dot(xu[...], w1a[...], preferred_element_type=jnp.float32)
    h = h + jnp.dot(xi[...], w1b[...], preferred_element_type=jnp.float32)
    h = jnp.maximum(h + b1[...], 0.0)
    h = jnp.maximum(jnp.dot(h, w2[...], preferred_element_type=jnp.float32) + b2[...], 0.0)
    h = jnp.maximum(jnp.dot(h, w3[...], preferred_element_type=jnp.float32) + b3[...], 0.0)
    mf = xum[...] * xim[...]
    logit = (jnp.dot(h, wah[...], preferred_element_type=jnp.float32)
             + jnp.dot(mf, wam[...], preferred_element_type=jnp.float32) + ba[...])
    out[...] = 1.0 / (1.0 + jnp.exp(-logit))


def _tc_mlp(xu, xi, xum, xim, W1, b1, W2, b2, W3, b3, Wa, ba):
    w1a = _blockdiag4(W1[:, :D].T)
    w1b = _blockdiag4(W1[:, D:].T)
    w2 = _blockdiag4(W2.T)
    w3 = _blockdiag4(W3.T)
    wah = _blockdiag4(Wa[:, :8].T)
    wam = _blockdiag4(Wa[:, 8:].T)
    b1t = jnp.tile(b1, 4).reshape(1, 128)
    b2t = jnp.tile(b2, 4).reshape(1, 64)
    b3t = jnp.tile(b3, 4).reshape(1, 32)
    row = lambda i: (i, 0)
    full = lambda i: (0, 0)
    grid = (B // 4) // BLK
    return pl.pallas_call(
        _mlp_body,
        grid=(grid,),
        in_specs=[
            pl.BlockSpec((BLK, 128), row),
            pl.BlockSpec((BLK, 128), row),
            pl.BlockSpec((BLK, 128), row),
            pl.BlockSpec((BLK, 128), row),
            pl.BlockSpec((128, 128), full),
            pl.BlockSpec((128, 128), full),
            pl.BlockSpec((1, 128), full),
            pl.BlockSpec((128, 64), full),
            pl.BlockSpec((1, 64), full),
            pl.BlockSpec((64, 32), full),
            pl.BlockSpec((1, 32), full),
            pl.BlockSpec((32, 4), full),
            pl.BlockSpec((128, 4), full),
            pl.BlockSpec((1, 1), full),
        ],
        out_specs=pl.BlockSpec((BLK, 4), row),
        out_shape=jax.ShapeDtypeStruct((B // 4, 4), jnp.float32),
    )(xu, xi, xum, xim, w1a, w1b, b1t, w2, b2t, w3, b3t, wah, wam,
      ba.reshape(1, 1))


def kernel(user_indices, item_indices, ratings, emb_user_mlp, emb_item_mlp,
           emb_user_mf, emb_item_mf, W1, b1, W2, b2, W3, b3, Wa, ba):
    xu, xi, xum, xim = _sc_gather4(
        user_indices.astype(jnp.int32), item_indices.astype(jnp.int32),
        emb_user_mlp.T, emb_item_mlp.T, emb_user_mf.T, emb_item_mf.T)
    packed = _tc_mlp(xu, xi, xum, xim, W1, b1, W2, b2, W3, b3, Wa, ba)
    rating = packed.reshape(B, 1)
    return (rating, ratings.astype(jnp.float32))


# speedup vs baseline: 3.8632x; 1.1180x over previous
import functools

import jax
import jax.numpy as jnp
from jax import lax
from jax.experimental import pallas as pl
from jax.experimental.pallas import tpu as pltpu
from jax.experimental.pallas import tpu_sc as plsc

B = 16384
D = 32
NC = 2
NS = 16
NW = NC * NS
ROWS_W = B // NW
TBK = 2
NSLOT = 4
NBAT = ROWS_W // TBK


def _splat(v, n=16):
    return jnp.broadcast_to(jnp.asarray(v, jnp.int32), (n,))


def _sc_gather4(uidx, iidx, t_umlp, t_imlp, t_umf, t_imf):
    mesh = plsc.VectorSubcoreMesh(
        core_axis_name="c", subcore_axis_name="s", num_cores=NC, num_subcores=NS
    )
    out_t = [jax.ShapeDtypeStruct((NW, 128, 128), jnp.float32) for _ in range(4)]

    @functools.partial(
        pl.kernel,
        out_type=out_t,
        mesh=mesh,
        compiler_params=pltpu.CompilerParams(
            use_tc_tiling_on_sc=True, needs_layout_passes=False),
        scratch_types=[
            pltpu.VMEM((2, ROWS_W // 16, 16), jnp.int32),
            pltpu.VMEM((NSLOT, 2, TBK, D, 128), jnp.float32),
            pltpu.VMEM((2, 128, 128), jnp.float32),
            pltpu.SemaphoreType.DMA((NSLOT,)),
        ],
    )
    def gather_kernel(uidx_hbm, iidx_hbm, umlp, imlp, umf, imf,
                      o_umlp, o_imlp, o_umf, o_imf,
                      idx_v, pan, outv, sems):
        wid = lax.axis_index("s") * NC + lax.axis_index("c")
        pltpu.sync_copy(uidx_hbm.at[wid], idx_v.at[0])
        pltpu.sync_copy(iidx_hbm.at[wid], idx_v.at[1])
        rows0 = lax.iota(jnp.int32, 16)
        rows1 = rows0 + 16

        def run_pass(p, tabs, outs):
            def sidx(r):
                vec = idx_v[p, lax.div(r, 16)]
                lane = lax.rem(r, 16)
                return jnp.max(jnp.where(rows0 == _splat(lane), vec, 0))

            def fire(g, slot):
                for l in range(TBK):
                    s = sidx(g * TBK + l)
                    c = pl.multiple_of(jnp.bitwise_and(s, -128), 128)
                    for t in range(2):
                        pltpu.async_copy(
                            tabs[t].at[:, pl.ds(c, 128)], pan.at[slot, t, l],
                            sems.at[slot])

            def drain_extract(g, slot, par):
                for t in range(2):
                    for l in range(TBK):
                        pltpu.make_async_copy(
                            tabs[t].at[:, pl.ds(0, 128)], pan.at[slot, t, l],
                            sems.at[slot]).wait()
                rq = lax.div(g, 2)
                for l in range(TBK):
                    col = _splat(jnp.bitwise_and(sidx(g * TBK + l), 127))
                    c0 = (2 * par + l) * D
                    for t in range(2):
                        v0 = plsc.load_gather(
                            pan, [_splat(slot), _splat(t), _splat(l), rows0, col])
                        v1 = plsc.load_gather(
                            pan, [_splat(slot), _splat(t), _splat(l), rows1, col])
                        outv[t, rq, pl.ds(c0, 16)] = v0
                        outv[t, rq, pl.ds(c0 + 16, 16)] = v1

            for s in range(NSLOT - 1):
                fire(s, s)

            def body(g4, carry):
                for k in range(NSLOT):
                    g = NSLOT * g4 + k
                    nxt = g + NSLOT - 1

                    @pl.when(nxt < NBAT)
                    def _():
                        fire(nxt, (k + NSLOT - 1) % NSLOT)

                    drain_extract(g, k, k % 2)
                return carry

            lax.fori_loop(0, NBAT // NSLOT, body, 0)
            for t in range(2):
                pltpu.sync_copy(outv.at[t], outs[t].at[wid])

        run_pass(0, (umlp, umf), (o_umlp, o_umf))
        run_pass(1, (imlp, imf), (o_imlp, o_imf))

    u2 = uidx.reshape(NW, ROWS_W // 16, 16)
    i2 = iidx.reshape(NW, ROWS_W // 16, 16)
    outs = gather_kernel(u2, i2, t_umlp, t_imlp, t_umf, t_imf)
    return [o.reshape(B // 4, 128) for o in outs]


def _blockdiag4(a):
    m, n = a.shape
    out = jnp.zeros((4 * m, 4 * n), jnp.float32)
    for j in range(4):
        out = out.at[j * m:(j + 1) * m, j * n:(j + 1) * n].set(a)
    return out


BLK = 1024


def _mlp_body(xu, xi, xum, xim, w1a, w1b, b1, w2, b2, w3, b3, wah, wam, ba, out):
    h = jnp.dot(xu[...], w1a[...], preferred_element_type=jnp.float32)
    h = h + jnp.dot(xi[...], w1b[...], preferred_element_type=jnp.float32)
    h = jnp.maximum(h + b1[...], 0.0)
    h = jnp.maximum(jnp.dot(h, w2[...], preferred_element_type=jnp.float32) + b2[...], 0.0)
    h = jnp.maximum(jnp.dot(h, w3[...], preferred_element_type=jnp.float32) + b3[...], 0.0)
    mf = xum[...] * xim[...]
    logit = (jnp.dot(h, wah[...], preferred_element_type=jnp.float32)
             + jnp.dot(mf, wam[...], preferred_element_type=jnp.float32) + ba[...])
    out[...] = 1.0 / (1.0 + jnp.exp(-logit))


def _tc_mlp(xu, xi, xum, xim, W1, b1, W2, b2, W3, b3, Wa, ba):
    w1a = _blockdiag4(W1[:, :D].T)
    w1b = _blockdiag4(W1[:, D:].T)
    w2 = _blockdiag4(W2.T)
    w3 = _blockdiag4(W3.T)
    wah = _blockdiag4(Wa[:, :8].T)
    wam = _blockdiag4(Wa[:, 8:].T)
    b1t = jnp.tile(b1, 4).reshape(1, 128)
    b2t = jnp.tile(b2, 4).reshape(1, 64)
    b3t = jnp.tile(b3, 4).reshape(1, 32)
    row = lambda i: (i, 0)
    full = lambda i: (0, 0)
    grid = (B // 4) // BLK
    return pl.pallas_call(
        _mlp_body,
        grid=(grid,),
        in_specs=[
            pl.BlockSpec((BLK, 128), row),
            pl.BlockSpec((BLK, 128), row),
            pl.BlockSpec((BLK, 128), row),
            pl.BlockSpec((BLK, 128), row),
            pl.BlockSpec((128, 128), full),
            pl.BlockSpec((128, 128), full),
            pl.BlockSpec((1, 128), full),
            pl.BlockSpec((128, 64), full),
            pl.BlockSpec((1, 64), full),
            pl.BlockSpec((64, 32), full),
            pl.BlockSpec((1, 32), full),
            pl.BlockSpec((32, 4), full),
            pl.BlockSpec((128, 4), full),
            pl.BlockSpec((1, 1), full),
        ],
        out_specs=pl.BlockSpec((BLK, 4), row),
        out_shape=jax.ShapeDtypeStruct((B // 4, 4), jnp.float32),
    )(xu, xi, xum, xim, w1a, w1b, b1t, w2, b2t, w3, b3t, wah, wam,
      ba.reshape(1, 1))


def kernel(user_indices, item_indices, ratings, emb_user_mlp, emb_item_mlp,
           emb_user_mf, emb_item_mf, W1, b1, W2, b2, W3, b3, Wa, ba):
    xu, xi, xum, xim = _sc_gather4(
        user_indices.astype(jnp.int32), item_indices.astype(jnp.int32),
        emb_user_mlp.T, emb_item_mlp.T, emb_user_mf.T, emb_item_mf.T)
    packed = _tc_mlp(xu, xi, xum, xim, W1, b1, W2, b2, W3, b3, Wa, ba)
    rating = packed.reshape(B, 1)
    return (rating, ratings.astype(jnp.float32))


# speedup vs baseline: 4.3499x; 1.1260x over previous
import functools

import jax
import jax.numpy as jnp
from jax import lax
from jax.experimental import pallas as pl
from jax.experimental.pallas import tpu as pltpu
from jax.experimental.pallas import tpu_sc as plsc

B = 16384
D = 32
NV = 1000000
NC = 2
NS = 16
NW = NC * NS
WPW = 256
CHW = 2
CHC = CHW * 128
NCH = WPW // CHW
CAP = 1024
CHSH = CHC.bit_length() - 1
WSH = (WPW * 128).bit_length() - 1
OUTR = B + 128


def _splat(v, n=16):
    return jnp.broadcast_to(jnp.asarray(v, jnp.int32), (n,))


def _sc_gather4(uidx, iidx, t_umlp, t_imlp, t_umf, t_imf):
    mesh = plsc.VectorSubcoreMesh(
        core_axis_name="c", subcore_axis_name="s", num_cores=NC, num_subcores=NS
    )
    out_t = [jax.ShapeDtypeStruct((OUTR, 128), jnp.float32) for _ in range(4)]

    @functools.partial(
        pl.kernel,
        out_type=out_t,
        mesh=mesh,
        compiler_params=pltpu.CompilerParams(
            use_tc_tiling_on_sc=True, needs_layout_passes=False),
        scratch_types=[
            pltpu.VMEM((B // 64, 16), jnp.int32),
            pltpu.VMEM((CAP,), jnp.int32),
            pltpu.VMEM((CAP,), jnp.int32),
            pltpu.VMEM((128,), jnp.int32),
            pltpu.VMEM((2, 2, D, CHC), jnp.float32),
            pltpu.VMEM((2, 128, 128), jnp.float32),
            pltpu.VMEM((2, 128), jnp.int32),
            pltpu.SemaphoreType.DMA((2,)),
            pltpu.SemaphoreType.DMA,
        ],
    )
    def gather_kernel(uidx_hbm, iidx_hbm, umlp, imlp, umf, imf,
                      o_umlp, o_imlp, o_umf, o_imf,
                      idx_all, idxl, posl, hitb, ring, stage, poss, sems,
                      osem):
        wid = lax.axis_index("s") * NC + lax.axis_index("c")
        lanes = lax.iota(jnp.int32, 16)
        rows1 = lanes + 16
        base_w = wid * (WPW * 128)
        nch = jnp.clip(lax.div(NV - base_w + CHC - 1, CHC), 0, NCH)

        def reset_poss():
            for k in range(8):
                poss[0, pl.ds(16 * k, 16)] = _splat(B) + lanes + 16 * k

        bc_max = (NV + 127) // 128 * 128 - CHC

        def chunk_base(c):
            return pl.multiple_of(jnp.minimum(base_w + c * CHC, bc_max), 128)

        def run_set(idx_hbm, tabs, outs):
            reset_poss()

            SEC = B // 64
            n = 0
            for sec in range(4):
                pltpu.sync_copy(idx_hbm.at[pl.ds(sec * SEC, SEC)], idx_all)

                def scan_body(j, off, sec=sec):
                    vec = idx_all[j]
                    m = lax.shift_right_logical(vec, WSH) == _splat(wid)
                    plsc.store_compressed(idxl.at[pl.ds(off, 16)], vec, mask=m)
                    plsc.store_compressed(
                        posl.at[pl.ds(off, 16)],
                        16 * (sec * SEC + j) + lanes, mask=m)
                    return off + plsc.all_reduce_population_count(m)[0]

                n = lax.fori_loop(0, SEC, scan_body, n)
            nv16 = lax.div(n + 15, 16)

            def fire(c, slot):
                bc = chunk_base(c)
                for t in range(2):
                    pltpu.async_copy(
                        tabs[t].at[:, pl.ds(bc, CHC)], ring.at[slot, t],
                        sems.at[slot])

            def flush(stcnt):
                for k in range(8):
                    lg = lanes + 16 * k
                    cur = poss[0, pl.ds(16 * k, 16)]
                    poss[0, pl.ds(16 * k, 16)] = jnp.where(
                        lg < _splat(stcnt), cur, _splat(B) + lg)
                for t in range(2):
                    pltpu.async_copy(stage.at[t], outs[t].at[poss.at[0]],
                                     osem).wait()

            def process(c, slot, stbase):
                for t in range(2):
                    pltpu.make_async_copy(
                        tabs[t].at[:, pl.ds(0, CHC)], ring.at[slot, t],
                        sems.at[slot]).wait()
                cglob = _splat(wid * NCH) + _splat(c)

                def sub_body(j, hc):
                    valid = (16 * j + lanes) < _splat(n)
                    vec = idxl[pl.ds(j * 16, 16)]
                    m = (lax.shift_right_logical(vec, CHSH) == cglob) & valid
                    pvec = posl[pl.ds(j * 16, 16)]
                    plsc.store_compressed(hitb.at[pl.ds(hc, 16)], vec, mask=m)
                    plsc.store_compressed(
                        poss.at[0, pl.ds(stbase + hc, 16)], pvec, mask=m)
                    return hc + plsc.all_reduce_population_count(m)[0]

                hc = lax.fori_loop(0, nv16, sub_body, 0)
                bc = chunk_base(c)

                def ext_body(h, carry):
                    hv = hitb[pl.ds(lax.mul(lax.div(h, 16), 16), 16)]
                    idxh = jnp.max(
                        jnp.where(lanes == _splat(lax.rem(h, 16)), hv, 0))
                    col = _splat(idxh - bc)
                    row = stbase + h
                    for t in range(2):
                        v0 = plsc.load_gather(
                            ring, [_splat(slot), _splat(t), lanes, col])
                        v1 = plsc.load_gather(
                            ring, [_splat(slot), _splat(t), rows1, col])
                        stage[t, row, pl.ds(0, 16)] = v0
                        stage[t, row, pl.ds(16, 16)] = v1
                    return carry

                lax.fori_loop(0, hc, ext_body, 0)
                return stbase + hc

            @pl.when(nch > 0)
            def _():
                fire(0, 0)

            def chunk_pair2(i, stbase):
                for k in range(2):
                    c = 2 * i + k

                    @pl.when(c + 1 < nch)
                    def _():
                        fire(c + 1, (k + 1) % 2)

                    in_range = c < nch
                    stcnt = lax.cond(
                        in_range,
                        lambda: process(c, k, stbase),
                        lambda: stbase)

                    do_flush = stcnt > 64

                    @pl.when(do_flush)
                    def _():
                        flush(stcnt)

                    stbase = jnp.where(do_flush, 0, stcnt)
                return stbase

            stbase = lax.fori_loop(0, NCH // 2, chunk_pair2, 0)

            @pl.when(stbase > 0)
            def _():
                flush(stbase)

        run_set(uidx_hbm, (umlp, umf), (o_umlp, o_umf))
        run_set(iidx_hbm, (imlp, imf), (o_imlp, o_imf))

    u2 = uidx.reshape(B // 16, 16)
    i2 = iidx.reshape(B // 16, 16)
    return gather_kernel(u2, i2, t_umlp, t_imlp, t_umf, t_imf)


BLK = 2048


def _mlp_body(xu, xi, xum, xim, w1a, w1b, b1, w2, b2, w3, b3, wah, wam, ba,
              out):
    ue = xu[...][:, :D]
    ie = xi[...][:, :D]
    mf = xum[...][:, :D] * xim[...][:, :D]
    h = jnp.dot(ue, w1a[...], preferred_element_type=jnp.float32)
    h = h + jnp.dot(ie, w1b[...], preferred_element_type=jnp.float32)
    h = jnp.maximum(h + b1[...], 0.0)
    h = jnp.maximum(jnp.dot(h, w2[...], preferred_element_type=jnp.float32) + b2[...], 0.0)
    h = jnp.maximum(jnp.dot(h, w3[...], preferred_element_type=jnp.float32) + b3[...], 0.0)
    logit = (jnp.dot(h, wah[...], preferred_element_type=jnp.float32)
             + jnp.dot(mf, wam[...], preferred_element_type=jnp.float32)
             + ba[...])
    out[...] = 1.0 / (1.0 + jnp.exp(-logit))


def _tc_mlp(xu, xi, xum, xim, W1, b1, W2, b2, W3, b3, Wa, ba):
    w1a = W1[:, :D].T
    w1b = W1[:, D:].T
    w2 = W2.T
    w3 = W3.T
    wah = Wa[:, :8].T
    wam = Wa[:, 8:].T
    row = lambda i: (i, 0)
    full = lambda i: (0, 0)
    grid = B // BLK
    return pl.pallas_call(
        _mlp_body,
        grid=(grid,),
        in_specs=[
            pl.BlockSpec((BLK, 128), row),
            pl.BlockSpec((BLK, 128), row),
            pl.BlockSpec((BLK, 128), row),
            pl.BlockSpec((BLK, 128), row),
            pl.BlockSpec((D, D), full),
            pl.BlockSpec((D, D), full),
            pl.BlockSpec((1, D), full),
            pl.BlockSpec((D, 16), full),
            pl.BlockSpec((1, 16), full),
            pl.BlockSpec((16, 8), full),
            pl.BlockSpec((1, 8), full),
            pl.BlockSpec((8, 1), full),
            pl.BlockSpec((D, 1), full),
            pl.BlockSpec((1, 1), full),
        ],
        out_specs=pl.BlockSpec((BLK, 1), row),
        out_shape=jax.ShapeDtypeStruct((B, 1), jnp.float32),
    )(xu, xi, xum, xim, w1a, w1b, b1.reshape(1, D), w2, b2.reshape(1, 16),
      w3, b3.reshape(1, 8), wah, wam, ba.reshape(1, 1))


def kernel(user_indices, item_indices, ratings, emb_user_mlp, emb_item_mlp,
           emb_user_mf, emb_item_mf, W1, b1, W2, b2, W3, b3, Wa, ba):
    xu, xi, xum, xim = _sc_gather4(
        user_indices.astype(jnp.int32), item_indices.astype(jnp.int32),
        emb_user_mlp.T, emb_item_mlp.T, emb_user_mf.T, emb_item_mf.T)
    rating = _tc_mlp(xu, xi, xum, xim, W1, b1, W2, b2, W3, b3, Wa, ba)
    return (rating, ratings.astype(jnp.float32))


# speedup vs baseline: 4.6316x; 1.0648x over previous
import functools

import jax
import jax.numpy as jnp
from jax import lax
from jax.experimental import pallas as pl
from jax.experimental.pallas import tpu as pltpu
from jax.experimental.pallas import tpu_sc as plsc

B = 16384
D = 32
NV = 1000000
NC = 2
NS = 16
NW = NC * NS
WPW = 256
CHW = 2
CHC = CHW * 128
NCH = WPW // CHW
CAP = 768
CHSH = CHC.bit_length() - 1
WSH = (WPW * 128).bit_length() - 1
OUTR = B + 128


def _splat(v, n=16):
    return jnp.broadcast_to(jnp.asarray(v, jnp.int32), (n,))


def _sc_gather4(uidx, iidx, t_umlp, t_imlp, t_umf, t_imf):
    mesh = plsc.VectorSubcoreMesh(
        core_axis_name="c", subcore_axis_name="s", num_cores=NC, num_subcores=NS
    )
    out_t = [jax.ShapeDtypeStruct((OUTR, 128), jnp.float32) for _ in range(4)]

    @functools.partial(
        pl.kernel,
        out_type=out_t,
        mesh=mesh,
        compiler_params=pltpu.CompilerParams(
            use_tc_tiling_on_sc=True, needs_layout_passes=False),
        scratch_types=[
            pltpu.VMEM((B // 128, 16), jnp.int32),
            pltpu.VMEM((CAP,), jnp.int32),
            pltpu.VMEM((CAP,), jnp.int32),
            pltpu.VMEM((128,), jnp.int32),
            pltpu.VMEM((4, 2, D, CHC), jnp.float32),
            pltpu.VMEM((2, 128, 128), jnp.float32),
            pltpu.VMEM((2, 128), jnp.int32),
            pltpu.SemaphoreType.DMA((4,)),
            pltpu.SemaphoreType.DMA,
        ],
    )
    def gather_kernel(uidx_hbm, iidx_hbm, umlp, imlp, umf, imf,
                      o_umlp, o_imlp, o_umf, o_imf,
                      idx_all, idxl, posl, hitb, ring, stage, poss, sems,
                      osem):
        wid = lax.axis_index("s") * NC + lax.axis_index("c")
        lanes = lax.iota(jnp.int32, 16)
        rows1 = lanes + 16
        base_w = wid * (WPW * 128)
        nch = jnp.clip(lax.div(NV - base_w + CHC - 1, CHC), 0, NCH)

        def reset_poss():
            for k in range(8):
                poss[0, pl.ds(16 * k, 16)] = _splat(B) + lanes + 16 * k

        bc_max = (NV + 127) // 128 * 128 - CHC

        def chunk_base(c):
            return pl.multiple_of(jnp.minimum(base_w + c * CHC, bc_max), 128)

        def run_set(idx_hbm, tabs, outs):
            reset_poss()

            SEC = B // 128
            n = 0
            for sec in range(8):
                pltpu.sync_copy(idx_hbm.at[pl.ds(sec * SEC, SEC)], idx_all)

                def scan_body(j, off, sec=sec):
                    vec = idx_all[j]
                    m = lax.shift_right_logical(vec, WSH) == _splat(wid)
                    plsc.store_compressed(idxl.at[pl.ds(off, 16)], vec, mask=m)
                    plsc.store_compressed(
                        posl.at[pl.ds(off, 16)],
                        16 * (sec * SEC + j) + lanes, mask=m)
                    return off + plsc.all_reduce_population_count(m)[0]

                n = lax.fori_loop(0, SEC, scan_body, n)
            nv16 = lax.div(n + 15, 16)

            def fire(c, slot):
                bc = chunk_base(c)
                for t in range(2):
                    pltpu.async_copy(
                        tabs[t].at[:, pl.ds(bc, CHC)], ring.at[slot, t],
                        sems.at[slot])

            def flush(stcnt):
                for k in range(8):
                    lg = lanes + 16 * k
                    cur = poss[0, pl.ds(16 * k, 16)]
                    poss[0, pl.ds(16 * k, 16)] = jnp.where(
                        lg < _splat(stcnt), cur, _splat(B) + lg)
                for t in range(2):
                    pltpu.async_copy(stage.at[t], outs[t].at[poss.at[0]],
                                     osem).wait()

            def process(c, slot, stbase):
                for t in range(2):
                    pltpu.make_async_copy(
                        tabs[t].at[:, pl.ds(0, CHC)], ring.at[slot, t],
                        sems.at[slot]).wait()
                cglob = _splat(wid * NCH) + _splat(c)

                def sub_body(j, hc):
                    valid = (16 * j + lanes) < _splat(n)
                    vec = idxl[pl.ds(j * 16, 16)]
                    m = (lax.shift_right_logical(vec, CHSH) == cglob) & valid
                    pvec = posl[pl.ds(j * 16, 16)]
                    plsc.store_compressed(hitb.at[pl.ds(hc, 16)], vec, mask=m)
                    plsc.store_compressed(
                        poss.at[0, pl.ds(stbase + hc, 16)], pvec, mask=m)
                    return hc + plsc.all_reduce_population_count(m)[0]

                hc = lax.fori_loop(0, nv16, sub_body, 0)
                bc = chunk_base(c)

                def ext_body(h, carry):
                    hv = hitb[pl.ds(lax.mul(lax.div(h, 16), 16), 16)]
                    idxh = jnp.max(
                        jnp.where(lanes == _splat(lax.rem(h, 16)), hv, 0))
                    col = _splat(idxh - bc)
                    row = stbase + h
                    for t in range(2):
                        v0 = plsc.load_gather(
                            ring, [_splat(slot), _splat(t), lanes, col])
                        v1 = plsc.load_gather(
                            ring, [_splat(slot), _splat(t), rows1, col])
                        stage[t, row, pl.ds(0, 16)] = v0
                        stage[t, row, pl.ds(16, 16)] = v1
                    return carry

                lax.fori_loop(0, hc, ext_body, 0)
                return stbase + hc

            for pre in range(3):
                @pl.when(pre < nch)
                def _(pre=pre):
                    fire(pre, pre)

            def chunk_pair2(i, stbase):
                for k in range(4):
                    c = 4 * i + k

                    @pl.when(c + 3 < nch)
                    def _():
                        fire(c + 3, (k + 3) % 4)

                    in_range = c < nch
                    stcnt = lax.cond(
                        in_range,
                        lambda: process(c, k, stbase),
                        lambda: stbase)

                    do_flush = stcnt > 64

                    @pl.when(do_flush)
                    def _():
                        flush(stcnt)

                    stbase = jnp.where(do_flush, 0, stcnt)
                return stbase

            stbase = lax.fori_loop(0, NCH // 4, chunk_pair2, 0)

            @pl.when(stbase > 0)
            def _():
                flush(stbase)

        run_set(uidx_hbm, (umlp, umf), (o_umlp, o_umf))
        run_set(iidx_hbm, (imlp, imf), (o_imlp, o_imf))

    u2 = uidx.reshape(B // 16, 16)
    i2 = iidx.reshape(B // 16, 16)
    return gather_kernel(u2, i2, t_umlp, t_imlp, t_umf, t_imf)


BLK = 2048


def _mlp_body(xu, xi, xum, xim, w1a, w1b, b1, w2, b2, w3, b3, wah, wam, ba,
              out):
    ue = xu[...][:, :D]
    ie = xi[...][:, :D]
    mf = xum[...][:, :D] * xim[...][:, :D]
    h = jnp.dot(ue, w1a[...], preferred_element_type=jnp.float32)
    h = h + jnp.dot(ie, w1b[...], preferred_element_type=jnp.float32)
    h = jnp.maximum(h + b1[...], 0.0)
    h = jnp.maximum(jnp.dot(h, w2[...], preferred_element_type=jnp.float32) + b2[...], 0.0)
    h = jnp.maximum(jnp.dot(h, w3[...], preferred_element_type=jnp.float32) + b3[...], 0.0)
    logit = (jnp.dot(h, wah[...], preferred_element_type=jnp.float32)
             + jnp.dot(mf, wam[...], preferred_element_type=jnp.float32)
             + ba[...])
    out[...] = 1.0 / (1.0 + jnp.exp(-logit))


def _tc_mlp(xu, xi, xum, xim, W1, b1, W2, b2, W3, b3, Wa, ba):
    w1a = W1[:, :D].T
    w1b = W1[:, D:].T
    w2 = W2.T
    w3 = W3.T
    wah = Wa[:, :8].T
    wam = Wa[:, 8:].T
    row = lambda i: (i, 0)
    full = lambda i: (0, 0)
    grid = B // BLK
    return pl.pallas_call(
        _mlp_body,
        grid=(grid,),
        in_specs=[
            pl.BlockSpec((BLK, 128), row),
            pl.BlockSpec((BLK, 128), row),
            pl.BlockSpec((BLK, 128), row),
            pl.BlockSpec((BLK, 128), row),
            pl.BlockSpec((D, D), full),
            pl.BlockSpec((D, D), full),
            pl.BlockSpec((1, D), full),
            pl.BlockSpec((D, 16), full),
            pl.BlockSpec((1, 16), full),
            pl.BlockSpec((16, 8), full),
            pl.BlockSpec((1, 8), full),
            pl.BlockSpec((8, 1), full),
            pl.BlockSpec((D, 1), full),
            pl.BlockSpec((1, 1), full),
        ],
        out_specs=pl.BlockSpec((BLK, 1), row),
        out_shape=jax.ShapeDtypeStruct((B, 1), jnp.float32),
    )(xu, xi, xum, xim, w1a, w1b, b1.reshape(1, D), w2, b2.reshape(1, 16),
      w3, b3.reshape(1, 8), wah, wam, ba.reshape(1, 1))


def kernel(user_indices, item_indices, ratings, emb_user_mlp, emb_item_mlp,
           emb_user_mf, emb_item_mf, W1, b1, W2, b2, W3, b3, Wa, ba):
    xu, xi, xum, xim = _sc_gather4(
        user_indices.astype(jnp.int32), item_indices.astype(jnp.int32),
        emb_user_mlp.T, emb_item_mlp.T, emb_user_mf.T, emb_item_mf.T)
    rating = _tc_mlp(xu, xi, xum, xim, W1, b1, W2, b2, W3, b3, Wa, ba)
    return (rating, ratings.astype(jnp.float32))
